# Initial kernel scaffold; baseline (speedup 1.0000x reference)
import functools

import jax
import jax.numpy as jnp
from jax import lax
from jax.experimental import pallas as pl
from jax.experimental.pallas import tpu as pltpu
from jax.experimental.pallas import tpu_sc as plsc

N_NODES = 10000
N_EDGES = 320000
UNITS = 128
DW = 144
N_PAD = 10016
NC, NS = 2, 16
NW = NC * NS
RPS = N_PAD // NS
LN = 128
CH = 80
E_PAD = NW * CH * LN
BN = 512


def _segsum_body(p_hbm, col_hbm, row_hbm, zeros_hbm, out_hbm,
                 colv, rowv, rowsv, acc_sh, sem):
    c = lax.axis_index("c")
    s = lax.axis_index("s")
    w = c * NS + s
    pltpu.sync_copy(zeros_hbm, acc_sh.at[pl.ds(s * RPS, RPS)])
    pltpu.sync_copy(col_hbm.at[w], colv)
    pltpu.sync_copy(row_hbm.at[w], rowv)
    plsc.subcore_barrier()

    @pl.loop(0, CH)
    def _chunk(j):
        pltpu.async_copy(p_hbm.at[colv.at[j]], rowsv, sem).wait()
        pltpu.sync_copy(rowsv, acc_sh.at[rowv.at[j]], add=True)

    plsc.subcore_barrier()
    pltpu.sync_copy(acc_sh.at[pl.ds(s * RPS, RPS)],
                    out_hbm.at[c, pl.ds(s * RPS, RPS)])


_segsum = pl.kernel(
    _segsum_body,
    out_type=jax.ShapeDtypeStruct((NC, N_PAD, DW), jnp.float32),
    mesh=plsc.VectorSubcoreMesh(core_axis_name="c", subcore_axis_name="s"),
    scratch_types=[
        pltpu.VMEM((CH, LN), jnp.int32),
        pltpu.VMEM((CH, LN), jnp.int32),
        pltpu.VMEM((LN, DW), jnp.float32),
        pltpu.VMEM_SHARED((N_PAD, DW), jnp.float32),
        pltpu.SemaphoreType.DMA,
    ],
)


def _proj_body(h_ref, ws_ref, wp_ref, e_ref, s_ref, p_ref):
    h = h_ref[...]
    s_ref[...] = jnp.dot(h, ws_ref[...], preferred_element_type=jnp.float32)
    p_ref[...] = (jnp.dot(h, wp_ref[...], preferred_element_type=jnp.float32)
                  + e_ref[...])


def _combine(s_ref, acc_ref, b_ref):
    a = acc_ref[0] + acc_ref[1]
    cnt = jnp.maximum(a[:, UNITS:UNITS + 1], 1.0)
    neigh = a[:, :UNITS] / cnt
    b = b_ref[...]
    h = jnp.concatenate([s_ref[...] + b[:, :UNITS], neigh + b[:, UNITS:]],
                        axis=1)
    return jnp.maximum(h, 0.0)


def _fused_body(s_ref, acc_ref, b_ref, ws_ref, wp_ref, e_ref, s_out, p_out):
    h = _combine(s_ref, acc_ref, b_ref)
    s_out[...] = jnp.dot(h, ws_ref[...], preferred_element_type=jnp.float32)
    p_out[...] = (jnp.dot(h, wp_ref[...], preferred_element_type=jnp.float32)
                  + e_ref[...])


def _head_body(s_ref, acc_ref, b_ref, wm1_ref, bm1_ref, wm2_ref, bm2_ref,
               o_ref):
    h = _combine(s_ref, acc_ref, b_ref)
    m = jnp.maximum(
        jnp.dot(h, wm1_ref[...], preferred_element_type=jnp.float32)
        + bm1_ref[...], 0.0)
    o_ref[...] = (jnp.dot(m, wm2_ref[...], preferred_element_type=jnp.float32)
                  + bm2_ref[...])


def _row_spec(width):
    return pl.BlockSpec((BN, width), lambda i: (i, 0))


def _full_spec(shape):
    nd = len(shape)
    return pl.BlockSpec(shape, lambda i: (0,) * nd)


_NB = pl.cdiv(N_NODES, BN)
_ACC_SPEC = pl.BlockSpec((NC, BN, DW), lambda i: (0, i, 0))


def _make_proj(din):
    return pl.pallas_call(
        _proj_body,
        grid=(_NB,),
        in_specs=[_row_spec(din), _full_spec((din, UNITS)),
                  _full_spec((din, DW)), _full_spec((1, DW))],
        out_specs=[_row_spec(UNITS), _row_spec(DW)],
        out_shape=[jax.ShapeDtypeStruct((N_NODES, UNITS), jnp.float32),
                   jax.ShapeDtypeStruct((N_NODES, DW), jnp.float32)],
    )


_proj1 = _make_proj(UNITS)

_fused = pl.pallas_call(
    _fused_body,
    grid=(_NB,),
    in_specs=[_row_spec(UNITS), _ACC_SPEC, _full_spec((1, 2 * UNITS)),
              _full_spec((2 * UNITS, UNITS)), _full_spec((2 * UNITS, DW)),
              _full_spec((1, DW))],
    out_specs=[_row_spec(UNITS), _row_spec(DW)],
    out_shape=[jax.ShapeDtypeStruct((N_NODES, UNITS), jnp.float32),
               jax.ShapeDtypeStruct((N_NODES, DW), jnp.float32)],
)

_head = pl.pallas_call(
    _head_body,
    grid=(_NB,),
    in_specs=[_row_spec(UNITS), _ACC_SPEC, _full_spec((1, 2 * UNITS)),
              _full_spec((2 * UNITS, 256)), _full_spec((1, 256)),
              _full_spec((256, 40)), _full_spec((1, 40))],
    out_specs=pl.BlockSpec((BN, 40), lambda i: (i, 0)),
    out_shape=jax.ShapeDtypeStruct((N_NODES, 40), jnp.float32),
)


def _pad_w(wn):
    return jnp.pad(wn, ((0, 0), (0, DW - UNITS)))


def kernel(x, edge_index, edge_weight, Ws1, Wn1, b1, Ws2, Wn2, b2,
           Ws3, Wn3, b3, Wm1, bm1, Wm2, bm2):
    del edge_weight
    row = edge_index[0]
    col = edge_index[1]
    pad = E_PAD - N_EDGES
    col_r = jnp.concatenate([col, jnp.zeros((pad,), jnp.int32)])
    col_r = col_r.reshape(NW, CH, LN)
    row_r = jnp.concatenate([row, jnp.full((pad,), N_NODES, jnp.int32)])
    row_r = row_r.reshape(NW, CH, LN)
    zeros_blk = jnp.zeros((RPS, DW), jnp.float32)
    e = jnp.zeros((1, DW), jnp.float32).at[0, UNITS].set(1.0)

    s1, p1 = _proj1(x, Ws1, _pad_w(Wn1), e)
    acc1 = _segsum(p1, col_r, row_r, zeros_blk)
    s2, p2 = _fused(s1, acc1, b1.reshape(1, -1), Ws2, _pad_w(Wn2), e)
    acc2 = _segsum(p2, col_r, row_r, zeros_blk)
    s3, p3 = _fused(s2, acc2, b2.reshape(1, -1), Ws3, _pad_w(Wn3), e)
    acc3 = _segsum(p3, col_r, row_r, zeros_blk)
    return _head(s3, acc3, b3.reshape(1, -1), Wm1, bm1.reshape(1, -1),
                 Wm2, bm2.reshape(1, -1))


# speedup vs baseline: 3.6650x; 3.6650x over previous
import functools

import jax
import jax.numpy as jnp
from jax import lax
from jax.experimental import pallas as pl
from jax.experimental.pallas import tpu as pltpu
from jax.experimental.pallas import tpu_sc as plsc

N_NODES = 10000
N_EDGES = 320000
U = 128
N_PAD = 10112
NC, NS = 2, 16
NW = NC * NS
RPS = N_PAD // NS
LN = 128
CH = 80
E_PAD = NW * CH * LN
BN = 512


def _segsum_body(p_hbm, col_hbm, row_hbm, zeros_hbm, out_hbm,
                 colv, rowv, rowsv, acc_sh, sem):
    c = lax.axis_index("c")
    s = lax.axis_index("s")
    w = c * NS + s
    pltpu.sync_copy(zeros_hbm, acc_sh.at[pl.ds(s * RPS, RPS)])
    pltpu.sync_copy(col_hbm.at[w], colv)
    pltpu.sync_copy(row_hbm.at[w], rowv)
    plsc.subcore_barrier()

    @pl.loop(0, CH)
    def _chunk(j):
        pltpu.async_copy(p_hbm.at[colv.at[j]], rowsv, sem).wait()
        pltpu.sync_copy(rowsv, acc_sh.at[rowv.at[j]], add=True)

    plsc.subcore_barrier()
    pltpu.sync_copy(acc_sh.at[pl.ds(s * RPS, RPS)],
                    out_hbm.at[c, pl.ds(s * RPS, RPS)])


def _count_body(row_hbm, ones_hbm, zeros_hbm, out_hbm,
                rowv, onesv, acc_sh):
    c = lax.axis_index("c")
    s = lax.axis_index("s")
    w = c * NS + s
    pltpu.sync_copy(zeros_hbm, acc_sh.at[pl.ds(s * RPS, RPS)])
    pltpu.sync_copy(row_hbm.at[w], rowv)
    pltpu.sync_copy(ones_hbm, onesv)
    plsc.subcore_barrier()

    @pl.loop(0, CH)
    def _chunk(j):
        pltpu.sync_copy(onesv, acc_sh.at[rowv.at[j]], add=True)

    plsc.subcore_barrier()
    pltpu.sync_copy(acc_sh.at[pl.ds(s * RPS, RPS)],
                    out_hbm.at[c, pl.ds(s * RPS, RPS)])


@functools.lru_cache(maxsize=None)
def _build_sc():
    mesh = plsc.VectorSubcoreMesh(core_axis_name="c", subcore_axis_name="s",
                                  num_cores=NC, num_subcores=NS)
    segsum = pl.kernel(
        _segsum_body,
        out_type=jax.ShapeDtypeStruct((NC, N_PAD, U), jnp.float32),
        mesh=mesh,
        scratch_types=[
            pltpu.VMEM((CH, LN), jnp.int32),
            pltpu.VMEM((CH, LN), jnp.int32),
            pltpu.VMEM((LN, U), jnp.float32),
            pltpu.VMEM_SHARED((N_PAD, U), jnp.float32),
            pltpu.SemaphoreType.DMA,
        ],
    )
    count = pl.kernel(
        _count_body,
        out_type=jax.ShapeDtypeStruct((NC, N_PAD, U), jnp.float32),
        mesh=mesh,
        scratch_types=[
            pltpu.VMEM((CH, LN), jnp.int32),
            pltpu.VMEM((LN, U), jnp.float32),
            pltpu.VMEM_SHARED((N_PAD, U), jnp.float32),
        ],
    )
    return segsum, count


def _segsum(p, col_r, row_r, zeros_blk):
    return _build_sc()[0](p, col_r, row_r, zeros_blk)


def _count(row_r, ones_blk, zeros_blk):
    return _build_sc()[1](row_r, ones_blk, zeros_blk)


def _proj_body(h_ref, ws_ref, wp_ref, s_ref, p_ref):
    h = h_ref[...]
    s_ref[...] = jnp.dot(h, ws_ref[...], preferred_element_type=jnp.float32)
    p_ref[...] = jnp.dot(h, wp_ref[...], preferred_element_type=jnp.float32)


def _combine(s_ref, acc_ref, cnt_ref, b_ref):
    a = acc_ref[0] + acc_ref[1]
    cnt = jnp.maximum(cnt_ref[0] + cnt_ref[1], 1.0)
    neigh = a / cnt
    b = b_ref[...]
    h = jnp.concatenate([s_ref[...] + b[:, :U], neigh + b[:, U:]], axis=1)
    return jnp.maximum(h, 0.0)


def _fused_body(s_ref, acc_ref, cnt_ref, b_ref, ws_ref, wp_ref, s_out, p_out):
    h = _combine(s_ref, acc_ref, cnt_ref, b_ref)
    s_out[...] = jnp.dot(h, ws_ref[...], preferred_element_type=jnp.float32)
    p_out[...] = jnp.dot(h, wp_ref[...], preferred_element_type=jnp.float32)


def _head_body(s_ref, acc_ref, cnt_ref, b_ref, wm1_ref, bm1_ref, wm2_ref,
               bm2_ref, o_ref):
    h = _combine(s_ref, acc_ref, cnt_ref, b_ref)
    m = jnp.maximum(
        jnp.dot(h, wm1_ref[...], preferred_element_type=jnp.float32)
        + bm1_ref[...], 0.0)
    o_ref[...] = (jnp.dot(m, wm2_ref[...], preferred_element_type=jnp.float32)
                  + bm2_ref[...])


def _row_spec(width):
    return pl.BlockSpec((BN, width), lambda i: (i, 0))


def _full_spec(shape):
    nd = len(shape)
    return pl.BlockSpec(shape, lambda i: (0,) * nd)


_NB = pl.cdiv(N_NODES, BN)
_ACC_SPEC = pl.BlockSpec((NC, BN, U), lambda i: (0, i, 0))
_SP_OUT = [_row_spec(U), _row_spec(U)]
_SP_SHAPE = [jax.ShapeDtypeStruct((N_NODES, U), jnp.float32),
             jax.ShapeDtypeStruct((N_NODES, U), jnp.float32)]


def _make_proj(din):
    return pl.pallas_call(
        _proj_body,
        grid=(_NB,),
        in_specs=[_row_spec(din), _full_spec((din, U)), _full_spec((din, U))],
        out_specs=_SP_OUT,
        out_shape=_SP_SHAPE,
    )


_proj1 = _make_proj(U)

_fused = pl.pallas_call(
    _fused_body,
    grid=(_NB,),
    in_specs=[_row_spec(U), _ACC_SPEC, _ACC_SPEC, _full_spec((1, 2 * U)),
              _full_spec((2 * U, U)), _full_spec((2 * U, U))],
    out_specs=_SP_OUT,
    out_shape=_SP_SHAPE,
)

_head = pl.pallas_call(
    _head_body,
    grid=(_NB,),
    in_specs=[_row_spec(U), _ACC_SPEC, _ACC_SPEC, _full_spec((1, 2 * U)),
              _full_spec((2 * U, 256)), _full_spec((1, 256)),
              _full_spec((256, 40)), _full_spec((1, 40))],
    out_specs=pl.BlockSpec((BN, 40), lambda i: (i, 0)),
    out_shape=jax.ShapeDtypeStruct((N_NODES, 40), jnp.float32),
)


def kernel(x, edge_index, edge_weight, Ws1, Wn1, b1, Ws2, Wn2, b2,
           Ws3, Wn3, b3, Wm1, bm1, Wm2, bm2):
    del edge_weight
    row = edge_index[0]
    col = edge_index[1]
    pad = E_PAD - N_EDGES
    col_r = jnp.concatenate([col, jnp.zeros((pad,), jnp.int32)])
    col_r = col_r.reshape(NW, CH, LN)
    row_r = jnp.concatenate([row, jnp.full((pad,), N_NODES, jnp.int32)])
    row_r = row_r.reshape(NW, CH, LN)
    zeros_blk = jnp.zeros((RPS, U), jnp.float32)
    ones_blk = jnp.ones((LN, U), jnp.float32)

    cnt = _count(row_r, ones_blk, zeros_blk)
    s1, p1 = _proj1(x, Ws1, Wn1)
    acc1 = _segsum(p1, col_r, row_r, zeros_blk)
    s2, p2 = _fused(s1, acc1, cnt, b1.reshape(1, -1), Ws2, Wn2)
    acc2 = _segsum(p2, col_r, row_r, zeros_blk)
    s3, p3 = _fused(s2, acc2, cnt, b2.reshape(1, -1), Ws3, Wn3)
    acc3 = _segsum(p3, col_r, row_r, zeros_blk)
    return _head(s3, acc3, cnt, b3.reshape(1, -1), Wm1, bm1.reshape(1, -1),
                 Wm2, bm2.reshape(1, -1))


# speedup vs baseline: 3.9390x; 1.0748x over previous
import functools

import jax
import jax.numpy as jnp
from jax import lax
from jax.experimental import pallas as pl
from jax.experimental.pallas import tpu as pltpu
from jax.experimental.pallas import tpu_sc as plsc

N_NODES = 10000
N_EDGES = 320000
U = 128
N_PAD = 10112
NC, NS = 2, 16
NW = NC * NS
RPS = N_PAD // NS
LN = 128
CH = 80
HCH = CH // 2
E_PAD = NW * CH * LN
BN = 512


def _segsum_body(p_hbm, col_hbm, row_hbm, zeros_hbm, out_hbm,
                 colv, rowv, rows_a, rows_b, acc_sh, sem_a, sem_b):
    c = lax.axis_index("c")
    s = lax.axis_index("s")
    w = c * NS + s
    pltpu.sync_copy(zeros_hbm, acc_sh.at[pl.ds(s * RPS, RPS)])
    plsc.subcore_barrier()

    for h in range(CH // HCH):
        pltpu.sync_copy(col_hbm.at[w, pl.ds(h * HCH, HCH)], colv)
        pltpu.sync_copy(row_hbm.at[w, pl.ds(h * HCH, HCH)], rowv)
        pltpu.async_copy(p_hbm.at[colv.at[0]], rows_a, sem_a)

        @pl.loop(0, HCH, step=2)
        def _chunk(j):
            pltpu.make_async_copy(p_hbm.at[colv.at[0]], rows_a, sem_a).wait()
            pltpu.async_copy(p_hbm.at[colv.at[j + 1]], rows_b, sem_b)
            pltpu.sync_copy(rows_a, acc_sh.at[rowv.at[j]], add=True)
            pltpu.make_async_copy(p_hbm.at[colv.at[0]], rows_b, sem_b).wait()

            @pl.when(j + 2 < HCH)
            def _():
                pltpu.async_copy(p_hbm.at[colv.at[j + 2]], rows_a, sem_a)

            pltpu.sync_copy(rows_b, acc_sh.at[rowv.at[j + 1]], add=True)

    plsc.subcore_barrier()
    pltpu.sync_copy(acc_sh.at[pl.ds(s * RPS, RPS)],
                    out_hbm.at[c, pl.ds(s * RPS, RPS)])


def _count_body(row_hbm, ones_hbm, zeros_hbm, out_hbm,
                rowv, onesv, acc_sh):
    c = lax.axis_index("c")
    s = lax.axis_index("s")
    w = c * NS + s
    pltpu.sync_copy(zeros_hbm, acc_sh.at[pl.ds(s * RPS, RPS)])
    pltpu.sync_copy(row_hbm.at[w], rowv)
    pltpu.sync_copy(ones_hbm, onesv)
    plsc.subcore_barrier()

    @pl.loop(0, CH)
    def _chunk(j):
        pltpu.sync_copy(onesv, acc_sh.at[rowv.at[j]], add=True)

    plsc.subcore_barrier()
    pltpu.sync_copy(acc_sh.at[pl.ds(s * RPS, RPS)],
                    out_hbm.at[c, pl.ds(s * RPS, RPS)])


@functools.lru_cache(maxsize=None)
def _build_sc():
    mesh = plsc.VectorSubcoreMesh(core_axis_name="c", subcore_axis_name="s",
                                  num_cores=NC, num_subcores=NS)
    segsum = pl.kernel(
        _segsum_body,
        out_type=jax.ShapeDtypeStruct((NC, N_PAD, U), jnp.float32),
        mesh=mesh,
        scratch_types=[
            pltpu.VMEM((HCH, LN), jnp.int32),
            pltpu.VMEM((HCH, LN), jnp.int32),
            pltpu.VMEM((LN, U), jnp.float32),
            pltpu.VMEM((LN, U), jnp.float32),
            pltpu.VMEM_SHARED((N_PAD, U), jnp.float32),
            pltpu.SemaphoreType.DMA,
            pltpu.SemaphoreType.DMA,
        ],
    )
    count = pl.kernel(
        _count_body,
        out_type=jax.ShapeDtypeStruct((NC, N_PAD, U), jnp.float32),
        mesh=mesh,
        scratch_types=[
            pltpu.VMEM((CH, LN), jnp.int32),
            pltpu.VMEM((LN, U), jnp.float32),
            pltpu.VMEM_SHARED((N_PAD, U), jnp.float32),
        ],
    )
    return segsum, count


def _segsum(p, col_r, row_r, zeros_blk):
    return _build_sc()[0](p, col_r, row_r, zeros_blk)


def _count(row_r, ones_blk, zeros_blk):
    return _build_sc()[1](row_r, ones_blk, zeros_blk)


def _proj_body(h_ref, ws_ref, wp_ref, s_ref, p_ref):
    h = h_ref[...]
    s_ref[...] = jnp.dot(h, ws_ref[...], preferred_element_type=jnp.float32)
    p_ref[...] = jnp.dot(h, wp_ref[...], preferred_element_type=jnp.float32)


def _combine(s_ref, acc_ref, cnt_ref, b_ref):
    a = acc_ref[0] + acc_ref[1]
    cnt = jnp.maximum(cnt_ref[0] + cnt_ref[1], 1.0)
    neigh = a / cnt
    b = b_ref[...]
    h = jnp.concatenate([s_ref[...] + b[:, :U], neigh + b[:, U:]], axis=1)
    return jnp.maximum(h, 0.0)


def _fused_body(s_ref, acc_ref, cnt_ref, b_ref, ws_ref, wp_ref, s_out, p_out):
    h = _combine(s_ref, acc_ref, cnt_ref, b_ref)
    s_out[...] = jnp.dot(h, ws_ref[...], preferred_element_type=jnp.float32)
    p_out[...] = jnp.dot(h, wp_ref[...], preferred_element_type=jnp.float32)


def _head_body(s_ref, acc_ref, cnt_ref, b_ref, wm1_ref, bm1_ref, wm2_ref,
               bm2_ref, o_ref):
    h = _combine(s_ref, acc_ref, cnt_ref, b_ref)
    m = jnp.maximum(
        jnp.dot(h, wm1_ref[...], preferred_element_type=jnp.float32)
        + bm1_ref[...], 0.0)
    o_ref[...] = (jnp.dot(m, wm2_ref[...], preferred_element_type=jnp.float32)
                  + bm2_ref[...])


def _row_spec(width):
    return pl.BlockSpec((BN, width), lambda i: (i, 0))


def _full_spec(shape):
    nd = len(shape)
    return pl.BlockSpec(shape, lambda i: (0,) * nd)


_NB = pl.cdiv(N_NODES, BN)
_ACC_SPEC = pl.BlockSpec((NC, BN, U), lambda i: (0, i, 0))
_SP_OUT = [_row_spec(U), _row_spec(U)]
_SP_SHAPE = [jax.ShapeDtypeStruct((N_NODES, U), jnp.float32),
             jax.ShapeDtypeStruct((N_NODES, U), jnp.float32)]


def _make_proj(din):
    return pl.pallas_call(
        _proj_body,
        grid=(_NB,),
        in_specs=[_row_spec(din), _full_spec((din, U)), _full_spec((din, U))],
        out_specs=_SP_OUT,
        out_shape=_SP_SHAPE,
    )


_proj1 = _make_proj(U)

_fused = pl.pallas_call(
    _fused_body,
    grid=(_NB,),
    in_specs=[_row_spec(U), _ACC_SPEC, _ACC_SPEC, _full_spec((1, 2 * U)),
              _full_spec((2 * U, U)), _full_spec((2 * U, U))],
    out_specs=_SP_OUT,
    out_shape=_SP_SHAPE,
)

_head = pl.pallas_call(
    _head_body,
    grid=(_NB,),
    in_specs=[_row_spec(U), _ACC_SPEC, _ACC_SPEC, _full_spec((1, 2 * U)),
              _full_spec((2 * U, 256)), _full_spec((1, 256)),
              _full_spec((256, 40)), _full_spec((1, 40))],
    out_specs=pl.BlockSpec((BN, 40), lambda i: (i, 0)),
    out_shape=jax.ShapeDtypeStruct((N_NODES, 40), jnp.float32),
)


def kernel(x, edge_index, edge_weight, Ws1, Wn1, b1, Ws2, Wn2, b2,
           Ws3, Wn3, b3, Wm1, bm1, Wm2, bm2):
    del edge_weight
    row = edge_index[0]
    col = edge_index[1]
    pad = E_PAD - N_EDGES
    col_r = jnp.concatenate([col, jnp.zeros((pad,), jnp.int32)])
    col_r = col_r.reshape(NW, CH, LN)
    row_r = jnp.concatenate([row, jnp.full((pad,), N_NODES, jnp.int32)])
    row_r = row_r.reshape(NW, CH, LN)
    zeros_blk = jnp.zeros((RPS, U), jnp.float32)
    ones_blk = jnp.ones((LN, U), jnp.float32)

    cnt = _count(row_r, ones_blk, zeros_blk)
    s1, p1 = _proj1(x, Ws1, Wn1)
    acc1 = _segsum(p1, col_r, row_r, zeros_blk)
    s2, p2 = _fused(s1, acc1, cnt, b1.reshape(1, -1), Ws2, Wn2)
    acc2 = _segsum(p2, col_r, row_r, zeros_blk)
    s3, p3 = _fused(s2, acc2, cnt, b2.reshape(1, -1), Ws3, Wn3)
    acc3 = _segsum(p3, col_r, row_r, zeros_blk)
    return _head(s3, acc3, cnt, b3.reshape(1, -1), Wm1, bm1.reshape(1, -1),
                 Wm2, bm2.reshape(1, -1))


# speedup vs baseline: 5.0144x; 1.2730x over previous
import functools

import jax
import jax.numpy as jnp
from jax import lax
from jax.experimental import pallas as pl
from jax.experimental.pallas import tpu as pltpu
from jax.experimental.pallas import tpu_sc as plsc

N_NODES = 10000
N_EDGES = 320000
U = 128
N_PAD = 10112
NC, NS = 2, 16
NW = NC * NS
RPS = N_PAD // NS
LN = 64
CH = 160
NBUF = 4
HCH = CH // 4
E_PAD = NW * CH * LN
BN = 512


def _segsum_body(p_hbm, col_hbm, row_hbm, zeros_hbm, out_hbm,
                 colv, rowv, rows, sems, acc_sh):
    c = lax.axis_index("c")
    s = lax.axis_index("s")
    w = c * NS + s
    pltpu.sync_copy(zeros_hbm, acc_sh.at[pl.ds(s * RPS, RPS)])
    plsc.subcore_barrier()

    for h in range(CH // HCH):
        pltpu.sync_copy(col_hbm.at[w, pl.ds(h * HCH, HCH)], colv)
        pltpu.sync_copy(row_hbm.at[w, pl.ds(h * HCH, HCH)], rowv)
        for b in range(NBUF):
            pltpu.async_copy(p_hbm.at[colv.at[b]], rows[b], sems[b])

        @pl.loop(0, HCH, step=NBUF)
        def _chunk(j):
            for b in range(NBUF):
                pltpu.make_async_copy(p_hbm.at[colv.at[0]], rows[b],
                                      sems[b]).wait()
                pltpu.sync_copy(rows[b], acc_sh.at[rowv.at[j + b]], add=True)

                @pl.when(j + b + NBUF < HCH)
                def _():
                    pltpu.async_copy(p_hbm.at[colv.at[j + b + NBUF]],
                                     rows[b], sems[b])

    plsc.subcore_barrier()
    pltpu.sync_copy(acc_sh.at[pl.ds(s * RPS, RPS)],
                    out_hbm.at[c, pl.ds(s * RPS, RPS)])


def _count_body(row_hbm, ones_hbm, zeros_hbm, out_hbm,
                rowv, onesv, acc_sh):
    c = lax.axis_index("c")
    s = lax.axis_index("s")
    w = c * NS + s
    pltpu.sync_copy(zeros_hbm, acc_sh.at[pl.ds(s * RPS, RPS)])
    pltpu.sync_copy(row_hbm.at[w], rowv)
    pltpu.sync_copy(ones_hbm, onesv)
    plsc.subcore_barrier()

    @pl.loop(0, CH)
    def _chunk(j):
        pltpu.sync_copy(onesv, acc_sh.at[rowv.at[j]], add=True)

    plsc.subcore_barrier()
    pltpu.sync_copy(acc_sh.at[pl.ds(s * RPS, RPS)],
                    out_hbm.at[c, pl.ds(s * RPS, RPS)])


@functools.lru_cache(maxsize=None)
def _build_sc():
    mesh = plsc.VectorSubcoreMesh(core_axis_name="c", subcore_axis_name="s",
                                  num_cores=NC, num_subcores=NS)
    segsum = pl.kernel(
        _segsum_body,
        out_type=jax.ShapeDtypeStruct((NC, N_PAD, U), jnp.float32),
        mesh=mesh,
        scratch_types=[
            pltpu.VMEM((HCH, LN), jnp.int32),
            pltpu.VMEM((HCH, LN), jnp.int32),
            [pltpu.VMEM((LN, U), jnp.float32) for _ in range(NBUF)],
            [pltpu.SemaphoreType.DMA for _ in range(NBUF)],
            pltpu.VMEM_SHARED((N_PAD, U), jnp.float32),
        ],
    )
    count = pl.kernel(
        _count_body,
        out_type=jax.ShapeDtypeStruct((NC, N_PAD, U), jnp.float32),
        mesh=mesh,
        scratch_types=[
            pltpu.VMEM((CH, LN), jnp.int32),
            pltpu.VMEM((LN, U), jnp.float32),
            pltpu.VMEM_SHARED((N_PAD, U), jnp.float32),
        ],
    )
    return segsum, count


def _segsum(p, col_r, row_r, zeros_blk):
    return _build_sc()[0](p, col_r, row_r, zeros_blk)


def _count(row_r, ones_blk, zeros_blk):
    return _build_sc()[1](row_r, ones_blk, zeros_blk)


def _proj_body(h_ref, ws_ref, wp_ref, s_ref, p_ref):
    h = h_ref[...]
    s_ref[...] = jnp.dot(h, ws_ref[...], preferred_element_type=jnp.float32)
    p_ref[...] = jnp.dot(h, wp_ref[...], preferred_element_type=jnp.float32)


def _combine(s_ref, acc_ref, cnt_ref, b_ref):
    a = acc_ref[0] + acc_ref[1]
    cnt = jnp.maximum(cnt_ref[0] + cnt_ref[1], 1.0)
    neigh = a / cnt
    b = b_ref[...]
    h = jnp.concatenate([s_ref[...] + b[:, :U], neigh + b[:, U:]], axis=1)
    return jnp.maximum(h, 0.0)


def _fused_body(s_ref, acc_ref, cnt_ref, b_ref, ws_ref, wp_ref, s_out, p_out):
    h = _combine(s_ref, acc_ref, cnt_ref, b_ref)
    s_out[...] = jnp.dot(h, ws_ref[...], preferred_element_type=jnp.float32)
    p_out[...] = jnp.dot(h, wp_ref[...], preferred_element_type=jnp.float32)


def _head_body(s_ref, acc_ref, cnt_ref, b_ref, wm1_ref, bm1_ref, wm2_ref,
               bm2_ref, o_ref):
    h = _combine(s_ref, acc_ref, cnt_ref, b_ref)
    m = jnp.maximum(
        jnp.dot(h, wm1_ref[...], preferred_element_type=jnp.float32)
        + bm1_ref[...], 0.0)
    o_ref[...] = (jnp.dot(m, wm2_ref[...], preferred_element_type=jnp.float32)
                  + bm2_ref[...])


def _row_spec(width):
    return pl.BlockSpec((BN, width), lambda i: (i, 0))


def _full_spec(shape):
    nd = len(shape)
    return pl.BlockSpec(shape, lambda i: (0,) * nd)


_NB = pl.cdiv(N_NODES, BN)
_ACC_SPEC = pl.BlockSpec((NC, BN, U), lambda i: (0, i, 0))
_SP_OUT = [_row_spec(U), _row_spec(U)]
_SP_SHAPE = [jax.ShapeDtypeStruct((N_NODES, U), jnp.float32),
             jax.ShapeDtypeStruct((N_NODES, U), jnp.float32)]


def _make_proj(din):
    return pl.pallas_call(
        _proj_body,
        grid=(_NB,),
        in_specs=[_row_spec(din), _full_spec((din, U)), _full_spec((din, U))],
        out_specs=_SP_OUT,
        out_shape=_SP_SHAPE,
    )


_proj1 = _make_proj(U)

_fused = pl.pallas_call(
    _fused_body,
    grid=(_NB,),
    in_specs=[_row_spec(U), _ACC_SPEC, _ACC_SPEC, _full_spec((1, 2 * U)),
              _full_spec((2 * U, U)), _full_spec((2 * U, U))],
    out_specs=_SP_OUT,
    out_shape=_SP_SHAPE,
)

_head = pl.pallas_call(
    _head_body,
    grid=(_NB,),
    in_specs=[_row_spec(U), _ACC_SPEC, _ACC_SPEC, _full_spec((1, 2 * U)),
              _full_spec((2 * U, 256)), _full_spec((1, 256)),
              _full_spec((256, 40)), _full_spec((1, 40))],
    out_specs=pl.BlockSpec((BN, 40), lambda i: (i, 0)),
    out_shape=jax.ShapeDtypeStruct((N_NODES, 40), jnp.float32),
)


def kernel(x, edge_index, edge_weight, Ws1, Wn1, b1, Ws2, Wn2, b2,
           Ws3, Wn3, b3, Wm1, bm1, Wm2, bm2):
    del edge_weight
    row = edge_index[0]
    col = edge_index[1]
    pad = E_PAD - N_EDGES
    col_r = jnp.concatenate([col, jnp.zeros((pad,), jnp.int32)])
    col_r = col_r.reshape(NW, CH, LN)
    row_r = jnp.concatenate([row, jnp.full((pad,), N_NODES, jnp.int32)])
    row_r = row_r.reshape(NW, CH, LN)
    zeros_blk = jnp.zeros((RPS, U), jnp.float32)
    ones_blk = jnp.ones((LN, U), jnp.float32)

    cnt = _count(row_r, ones_blk, zeros_blk)
    s1, p1 = _proj1(x, Ws1, Wn1)
    acc1 = _segsum(p1, col_r, row_r, zeros_blk)
    s2, p2 = _fused(s1, acc1, cnt, b1.reshape(1, -1), Ws2, Wn2)
    acc2 = _segsum(p2, col_r, row_r, zeros_blk)
    s3, p3 = _fused(s2, acc2, cnt, b2.reshape(1, -1), Ws3, Wn3)
    acc3 = _segsum(p3, col_r, row_r, zeros_blk)
    return _head(s3, acc3, cnt, b3.reshape(1, -1), Wm1, bm1.reshape(1, -1),
                 Wm2, bm2.reshape(1, -1))
